# Initial kernel scaffold; baseline (speedup 1.0000x reference)
import functools

import jax
import jax.numpy as jnp
from jax import lax
from jax.experimental import pallas as pl
from jax.experimental.pallas import tpu as pltpu
from jax.experimental.pallas import tpu_sc as plsc

NC = 2
NS = 16
K = 80


def _mesh():
    return plsc.VectorSubcoreMesh(core_axis_name="c", subcore_axis_name="s")


def _deg_call(dst, init):
    n = init.shape[0]
    e = dst.shape[0]
    e_per = e // (NC * NS)
    iters = e_per // K
    n_chunk = 1000
    n_tiles_io = n // n_chunk

    @functools.partial(
        pl.kernel,
        out_type=jax.ShapeDtypeStruct((NC, n), jnp.float32),
        mesh=_mesh(),
        scratch_types=[
            pltpu.VMEM((K,), jnp.int32),
            pltpu.VMEM((K,), jnp.float32),
            pltpu.VMEM_SHARED((n,), jnp.float32),
        ],
    )
    def deg_k(dst_hbm, init_hbm, out_hbm, idx_v, ones_v, acc_sh):
        c = lax.axis_index("c")
        s = lax.axis_index("s")

        @pl.when(s < n_tiles_io)
        def _():
            pltpu.sync_copy(init_hbm.at[pl.ds(s * n_chunk, n_chunk)],
                            acc_sh.at[pl.ds(s * n_chunk, n_chunk)])

        for i in range(K // 16):
            ones_v[pl.ds(i * 16, 16)] = jnp.full((16,), 1.0, jnp.float32)
        plsc.subcore_barrier()

        base = (c * NS + s) * e_per

        def body(i, carry):
            off = base + i * K
            pltpu.sync_copy(dst_hbm.at[pl.ds(off, K)], idx_v)
            pltpu.sync_copy(ones_v, acc_sh.at[idx_v], add=True)
            return carry

        lax.fori_loop(0, iters, body, 0)
        plsc.subcore_barrier()

        @pl.when(s < n_tiles_io)
        def _():
            pltpu.sync_copy(acc_sh.at[pl.ds(s * n_chunk, n_chunk)],
                            out_hbm.at[c, pl.ds(s * n_chunk, n_chunk)])

    return deg_k(dst, init)


def _agg_call(g, src, dst, zeros):
    n, w = g.shape
    e = src.shape[0]
    e_per = e // (NC * NS)
    iters = e_per // K
    rows_per = n // NS

    @functools.partial(
        pl.kernel,
        out_type=jax.ShapeDtypeStruct((NC, n, w), jnp.float32),
        mesh=_mesh(),
        scratch_types=[
            pltpu.VMEM((K,), jnp.int32),
            pltpu.VMEM((K,), jnp.int32),
            pltpu.VMEM((K, w), jnp.float32),
            pltpu.VMEM_SHARED((n, w), jnp.float32),
            pltpu.SemaphoreType.DMA,
        ],
    )
    def agg_k(g_hbm, src_hbm, dst_hbm, z_hbm, out_hbm,
              idxs_v, idxd_v, rows_v, acc_sh, sem):
        c = lax.axis_index("c")
        s = lax.axis_index("s")
        pltpu.sync_copy(z_hbm.at[pl.ds(s * rows_per, rows_per)],
                        acc_sh.at[pl.ds(s * rows_per, rows_per)])
        plsc.subcore_barrier()

        base = (c * NS + s) * e_per

        def body(i, carry):
            off = base + i * K
            pltpu.sync_copy(src_hbm.at[pl.ds(off, K)], idxs_v)
            pltpu.sync_copy(dst_hbm.at[pl.ds(off, K)], idxd_v)
            pltpu.async_copy(g_hbm.at[idxs_v], rows_v, sem).wait()
            pltpu.sync_copy(rows_v, acc_sh.at[idxd_v], add=True)
            return carry

        lax.fori_loop(0, iters, body, 0)
        plsc.subcore_barrier()
        pltpu.sync_copy(acc_sh.at[pl.ds(s * rows_per, rows_per)],
                        out_hbm.at[c, pl.ds(s * rows_per, rows_per)])

    return agg_k(g, src, dst, zeros)


_BR = 1000


def _tc_first(x, w0, dega, degb):
    n, d = x.shape
    h = w0.shape[1]

    def body(x_ref, w_ref, da_ref, db_ref, h_ref, g_ref, dv_ref):
        deg = da_ref[...] + db_ref[...]
        dinv = lax.rsqrt(deg)
        hh = jnp.dot(x_ref[...], w_ref[...], preferred_element_type=jnp.float32)
        h_ref[...] = hh
        g_ref[...] = hh * dinv
        dv_ref[...] = dinv

    return pl.pallas_call(
        body,
        grid=(n // _BR,),
        in_specs=[
            pl.BlockSpec((_BR, d), lambda i: (i, 0)),
            pl.BlockSpec((d, h), lambda i: (0, 0)),
            pl.BlockSpec((_BR, 1), lambda i: (i, 0)),
            pl.BlockSpec((_BR, 1), lambda i: (i, 0)),
        ],
        out_specs=[
            pl.BlockSpec((_BR, h), lambda i: (i, 0)),
            pl.BlockSpec((_BR, h), lambda i: (i, 0)),
            pl.BlockSpec((_BR, 1), lambda i: (i, 0)),
        ],
        out_shape=[
            jax.ShapeDtypeStruct((n, h), jnp.float32),
            jax.ShapeDtypeStruct((n, h), jnp.float32),
            jax.ShapeDtypeStruct((n, 1), jnp.float32),
        ],
    )(x, w0, dega, degb)


def _tc_mid(acca, accb, hprev, dinv, b, w):
    n, hp = hprev.shape
    hn = w.shape[1]

    def body(aa_ref, ab_ref, hp_ref, dv_ref, b_ref, w_ref, h_ref, g_ref):
        dinv = dv_ref[...]
        acc = aa_ref[...] + ab_ref[...]
        xn = acc * dinv + (2.0 * dinv * dinv) * hp_ref[...] + b_ref[...]
        xn = jnp.maximum(xn, 0.0)
        hh = jnp.dot(xn, w_ref[...], preferred_element_type=jnp.float32)
        h_ref[...] = hh
        g_ref[...] = hh * dinv

    return pl.pallas_call(
        body,
        grid=(n // _BR,),
        in_specs=[
            pl.BlockSpec((_BR, hp), lambda i: (i, 0)),
            pl.BlockSpec((_BR, hp), lambda i: (i, 0)),
            pl.BlockSpec((_BR, hp), lambda i: (i, 0)),
            pl.BlockSpec((_BR, 1), lambda i: (i, 0)),
            pl.BlockSpec((1, hp), lambda i: (0, 0)),
            pl.BlockSpec((hp, hn), lambda i: (0, 0)),
        ],
        out_specs=[
            pl.BlockSpec((_BR, hn), lambda i: (i, 0)),
            pl.BlockSpec((_BR, hn), lambda i: (i, 0)),
        ],
        out_shape=[
            jax.ShapeDtypeStruct((n, hn), jnp.float32),
            jax.ShapeDtypeStruct((n, hn), jnp.float32),
        ],
    )(acca, accb, hprev, dinv, b, w)


def _tc_final(acca, accb, hprev, dinv, b):
    n, hp = hprev.shape

    def body(aa_ref, ab_ref, hp_ref, dv_ref, b_ref, o_ref):
        dinv = dv_ref[...]
        acc = aa_ref[...] + ab_ref[...]
        xn = acc * dinv + (2.0 * dinv * dinv) * hp_ref[...] + b_ref[...]
        o_ref[...] = jnp.tanh(xn)

    return pl.pallas_call(
        body,
        grid=(n // _BR,),
        in_specs=[
            pl.BlockSpec((_BR, hp), lambda i: (i, 0)),
            pl.BlockSpec((_BR, hp), lambda i: (i, 0)),
            pl.BlockSpec((_BR, hp), lambda i: (i, 0)),
            pl.BlockSpec((_BR, 1), lambda i: (i, 0)),
            pl.BlockSpec((1, hp), lambda i: (0, 0)),
        ],
        out_specs=pl.BlockSpec((_BR, hp), lambda i: (i, 0)),
        out_shape=jax.ShapeDtypeStruct((n, hp), jnp.float32),
    )(acca, accb, hprev, dinv, b)


def kernel(x, edge_index, W0, b0, W1, b1, W2, b2):
    n, d = x.shape
    h = W0.shape[1]
    c_out = W2.shape[1]
    c_pad = 64

    src = edge_index[0].astype(jnp.int32)
    dst = edge_index[1].astype(jnp.int32)

    init1 = jnp.full((n,), 1.0, jnp.float32)
    zeros_h = jnp.zeros((n, h), jnp.float32)
    zeros_c = jnp.zeros((n, c_pad), jnp.float32)
    b0r = b0.reshape(1, h)
    b1r = b1.reshape(1, h)
    w2p = jnp.zeros((h, c_pad), jnp.float32).at[:, :c_out].set(W2)
    b2p = jnp.zeros((1, c_pad), jnp.float32).at[0, :c_out].set(b2)

    degp = _deg_call(dst, init1)
    dega = degp[0].reshape(n, 1)
    degb = degp[1].reshape(n, 1)

    h0, g0, dinv = _tc_first(x, W0, dega, degb)
    acc0 = _agg_call(g0, src, dst, zeros_h)
    h1, g1 = _tc_mid(acc0[0], acc0[1], h0, dinv, b0r, W1)
    acc1 = _agg_call(g1, src, dst, zeros_h)
    h2, g2 = _tc_mid(acc1[0], acc1[1], h1, dinv, b1r, w2p)
    acc2 = _agg_call(g2, src, dst, zeros_c)
    out = _tc_final(acc2[0], acc2[1], h2, dinv, b2p)
    return out[:, :c_out]


# speedup vs baseline: 10.7496x; 10.7496x over previous
import functools

import jax
import jax.numpy as jnp
from jax import lax
from jax.experimental import pallas as pl
from jax.experimental.pallas import tpu as pltpu
from jax.experimental.pallas import tpu_sc as plsc

NC = 2
NS = 16
K = 80


def _mesh():
    return plsc.VectorSubcoreMesh(core_axis_name="c", subcore_axis_name="s")


def _deg_call(dst, n):
    e = dst.shape[0]
    e_per = e // (NC * NS)
    iters = e_per // K
    n_chunk = 1000
    n_tiles_io = n // n_chunk

    @functools.partial(
        pl.kernel,
        out_type=jax.ShapeDtypeStruct((NC * n,), jnp.float32),
        mesh=_mesh(),
        scratch_types=[
            pltpu.VMEM((K,), jnp.int32),
            pltpu.VMEM((K,), jnp.float32),
            pltpu.VMEM((n_chunk,), jnp.float32),
            pltpu.VMEM_SHARED((n,), jnp.float32),
        ],
    )
    def deg_k(dst_hbm, out_hbm, idx_v, ones_v, stage_v, acc_sh):
        c = lax.axis_index("c")
        s = lax.axis_index("s")

        for i in range(K // 16):
            ones_v[pl.ds(i * 16, 16)] = jnp.full((16,), 1.0, jnp.float32)

        @pl.when(s < n_tiles_io)
        def _():
            zero = jnp.zeros((16,), jnp.float32)
            for i in range(n_chunk // 16):
                stage_v[pl.ds(i * 16, 16)] = zero
            if n_chunk % 16:
                stage_v[pl.ds(n_chunk - 16, 16)] = zero
            off = pl.multiple_of(s * n_chunk, 8)
            pltpu.sync_copy(stage_v, acc_sh.at[pl.ds(off, n_chunk)])

        plsc.subcore_barrier()

        base = (c * NS + s) * e_per

        def body(i, carry):
            off = pl.multiple_of(base + i * K, 8)
            pltpu.sync_copy(dst_hbm.at[pl.ds(off, K)], idx_v)
            pltpu.sync_copy(ones_v, acc_sh.at[idx_v], add=True)
            return carry

        lax.fori_loop(0, iters, body, 0)
        plsc.subcore_barrier()

        @pl.when(s < n_tiles_io)
        def _():
            off = pl.multiple_of(s * n_chunk, 8)
            oout = pl.multiple_of(c * n + s * n_chunk, 8)
            pltpu.sync_copy(acc_sh.at[pl.ds(off, n_chunk)], stage_v)
            pltpu.sync_copy(stage_v, out_hbm.at[pl.ds(oout, n_chunk)])

    return deg_k(dst)


def _agg_call(g, src, dst):
    n, w = g.shape
    e = src.shape[0]
    e_per = e // (NC * NS)
    iters = e_per // K
    st_rows = 200
    n_chunk = 1000
    n_tiles_io = n // n_chunk
    st_per = n_chunk // st_rows

    @functools.partial(
        pl.kernel,
        out_type=jax.ShapeDtypeStruct((NC, n, w), jnp.float32),
        mesh=_mesh(),
        scratch_types=[
            pltpu.VMEM((K,), jnp.int32),
            pltpu.VMEM((K,), jnp.int32),
            pltpu.VMEM((K, w), jnp.float32),
            pltpu.VMEM((st_rows, w), jnp.float32),
            pltpu.VMEM_SHARED((n, w), jnp.float32),
            pltpu.SemaphoreType.DMA,
        ],
    )
    def agg_k(g_hbm, src_hbm, dst_hbm, out_hbm,
              idxs_v, idxd_v, rows_v, stage_v, acc_sh, sem):
        c = lax.axis_index("c")
        s = lax.axis_index("s")

        @pl.when(s < n_tiles_io)
        def _():
            zero = jnp.zeros((16,), jnp.float32)

            def zrow(i, carry):
                for j in range(w // 16):
                    stage_v[i, pl.ds(j * 16, 16)] = zero
                return carry

            lax.fori_loop(0, st_rows, zrow, 0)
            for j in range(st_per):
                off = pl.multiple_of(s * n_chunk + j * st_rows, 8)
                pltpu.sync_copy(stage_v, acc_sh.at[pl.ds(off, st_rows)])

        plsc.subcore_barrier()

        base = (c * NS + s) * e_per

        def body(i, carry):
            off = pl.multiple_of(base + i * K, 8)
            pltpu.sync_copy(src_hbm.at[pl.ds(off, K)], idxs_v)
            pltpu.sync_copy(dst_hbm.at[pl.ds(off, K)], idxd_v)
            pltpu.async_copy(g_hbm.at[idxs_v], rows_v, sem).wait()
            pltpu.sync_copy(rows_v, acc_sh.at[idxd_v], add=True)
            return carry

        lax.fori_loop(0, iters, body, 0)
        plsc.subcore_barrier()

        @pl.when(s < n_tiles_io)
        def _():
            for j in range(st_per):
                off = pl.multiple_of(s * n_chunk + j * st_rows, 8)
                pltpu.sync_copy(acc_sh.at[pl.ds(off, st_rows)], stage_v)
                pltpu.sync_copy(stage_v, out_hbm.at[c, pl.ds(off, st_rows)])

    return agg_k(g, src, dst)


_BR = 1000


def _tc_first(x, w0, dega, degb):
    n, d = x.shape
    h = w0.shape[1]

    def body(x_ref, w_ref, da_ref, db_ref, h_ref, g_ref, dv_ref):
        deg = da_ref[...] + db_ref[...] + 2.0
        dinv = lax.rsqrt(deg)
        hh = jnp.dot(x_ref[...], w_ref[...], preferred_element_type=jnp.float32)
        h_ref[...] = hh
        g_ref[...] = hh * dinv
        dv_ref[...] = dinv

    return pl.pallas_call(
        body,
        grid=(n // _BR,),
        in_specs=[
            pl.BlockSpec((_BR, d), lambda i: (i, 0)),
            pl.BlockSpec((d, h), lambda i: (0, 0)),
            pl.BlockSpec((_BR, 1), lambda i: (i, 0)),
            pl.BlockSpec((_BR, 1), lambda i: (i, 0)),
        ],
        out_specs=[
            pl.BlockSpec((_BR, h), lambda i: (i, 0)),
            pl.BlockSpec((_BR, h), lambda i: (i, 0)),
            pl.BlockSpec((_BR, 1), lambda i: (i, 0)),
        ],
        out_shape=[
            jax.ShapeDtypeStruct((n, h), jnp.float32),
            jax.ShapeDtypeStruct((n, h), jnp.float32),
            jax.ShapeDtypeStruct((n, 1), jnp.float32),
        ],
    )(x, w0, dega, degb)


def _tc_mid(acca, accb, hprev, dinv, b, w):
    n, hp = hprev.shape
    hn = w.shape[1]

    def body(aa_ref, ab_ref, hp_ref, dv_ref, b_ref, w_ref, h_ref, g_ref):
        dinv = dv_ref[...]
        acc = aa_ref[...] + ab_ref[...]
        xn = acc * dinv + (2.0 * dinv * dinv) * hp_ref[...] + b_ref[...]
        xn = jnp.maximum(xn, 0.0)
        hh = jnp.dot(xn, w_ref[...], preferred_element_type=jnp.float32)
        h_ref[...] = hh
        g_ref[...] = hh * dinv

    return pl.pallas_call(
        body,
        grid=(n // _BR,),
        in_specs=[
            pl.BlockSpec((_BR, hp), lambda i: (i, 0)),
            pl.BlockSpec((_BR, hp), lambda i: (i, 0)),
            pl.BlockSpec((_BR, hp), lambda i: (i, 0)),
            pl.BlockSpec((_BR, 1), lambda i: (i, 0)),
            pl.BlockSpec((1, hp), lambda i: (0, 0)),
            pl.BlockSpec((hp, hn), lambda i: (0, 0)),
        ],
        out_specs=[
            pl.BlockSpec((_BR, hn), lambda i: (i, 0)),
            pl.BlockSpec((_BR, hn), lambda i: (i, 0)),
        ],
        out_shape=[
            jax.ShapeDtypeStruct((n, hn), jnp.float32),
            jax.ShapeDtypeStruct((n, hn), jnp.float32),
        ],
    )(acca, accb, hprev, dinv, b, w)


def _tc_final(acca, accb, hprev, dinv, b):
    n, hp = hprev.shape

    def body(aa_ref, ab_ref, hp_ref, dv_ref, b_ref, o_ref):
        dinv = dv_ref[...]
        acc = aa_ref[...] + ab_ref[...]
        xn = acc * dinv + (2.0 * dinv * dinv) * hp_ref[...] + b_ref[...]
        o_ref[...] = jnp.tanh(xn)

    return pl.pallas_call(
        body,
        grid=(n // _BR,),
        in_specs=[
            pl.BlockSpec((_BR, hp), lambda i: (i, 0)),
            pl.BlockSpec((_BR, hp), lambda i: (i, 0)),
            pl.BlockSpec((_BR, hp), lambda i: (i, 0)),
            pl.BlockSpec((_BR, 1), lambda i: (i, 0)),
            pl.BlockSpec((1, hp), lambda i: (0, 0)),
        ],
        out_specs=pl.BlockSpec((_BR, hp), lambda i: (i, 0)),
        out_shape=jax.ShapeDtypeStruct((n, hp), jnp.float32),
    )(acca, accb, hprev, dinv, b)


def kernel(x, edge_index, W0, b0, W1, b1, W2, b2):
    n, d = x.shape
    h = W0.shape[1]
    c_out = W2.shape[1]
    c_pad = 128

    src = edge_index[0].astype(jnp.int32)
    dst = edge_index[1].astype(jnp.int32)

    b0r = b0.reshape(1, h)
    b1r = b1.reshape(1, h)
    w2p = jnp.zeros((h, c_pad), jnp.float32).at[:, :c_out].set(W2)
    b2p = jnp.zeros((1, c_pad), jnp.float32).at[0, :c_out].set(b2)

    degp = _deg_call(dst, n)
    dega = degp[:n].reshape(n, 1)
    degb = degp[n:].reshape(n, 1)

    h0, g0, dinv = _tc_first(x, W0, dega, degb)
    acc0 = _agg_call(g0, src, dst)
    h1, g1 = _tc_mid(acc0[0], acc0[1], h0, dinv, b0r, W1)
    acc1 = _agg_call(g1, src, dst)
    h2, g2 = _tc_mid(acc1[0], acc1[1], h1, dinv, b1r, w2p)
    acc2 = _agg_call(g2, src, dst)
    out = _tc_final(acc2[0], acc2[1], h2, dinv, b2p)
    return out[:, :c_out]


# speedup vs baseline: 18.8081x; 1.7497x over previous
import functools

import jax
import jax.numpy as jnp
from jax import lax
from jax.experimental import pallas as pl
from jax.experimental.pallas import tpu as pltpu
from jax.experimental.pallas import tpu_sc as plsc

NC = 2
NS = 16
K = 80


def _mesh():
    return plsc.VectorSubcoreMesh(core_axis_name="c", subcore_axis_name="s")


def _deg_call(dst, n):
    e = dst.shape[0]
    e_per = e // (NC * NS)
    iters = e_per // K
    n_chunk = 1000
    n_tiles_io = n // n_chunk

    @functools.partial(
        pl.kernel,
        out_type=jax.ShapeDtypeStruct((NC * n,), jnp.float32),
        mesh=_mesh(),
        scratch_types=[
            pltpu.VMEM((K,), jnp.int32),
            pltpu.VMEM((K,), jnp.float32),
            pltpu.VMEM((n_chunk,), jnp.float32),
            pltpu.VMEM_SHARED((n,), jnp.float32),
        ],
    )
    def deg_k(dst_hbm, out_hbm, idx_v, ones_v, stage_v, acc_sh):
        c = lax.axis_index("c")
        s = lax.axis_index("s")

        for i in range(K // 16):
            ones_v[pl.ds(i * 16, 16)] = jnp.full((16,), 1.0, jnp.float32)

        @pl.when(s < n_tiles_io)
        def _():
            zero = jnp.zeros((16,), jnp.float32)
            for i in range(n_chunk // 16):
                stage_v[pl.ds(i * 16, 16)] = zero
            if n_chunk % 16:
                stage_v[pl.ds(n_chunk - 16, 16)] = zero
            off = pl.multiple_of(s * n_chunk, 8)
            pltpu.sync_copy(stage_v, acc_sh.at[pl.ds(off, n_chunk)])

        plsc.subcore_barrier()

        base = (c * NS + s) * e_per

        def body(i, carry):
            off = pl.multiple_of(base + i * K, 8)
            pltpu.sync_copy(dst_hbm.at[pl.ds(off, K)], idx_v)
            pltpu.sync_copy(ones_v, acc_sh.at[idx_v], add=True)
            return carry

        lax.fori_loop(0, iters, body, 0)
        plsc.subcore_barrier()

        @pl.when(s < n_tiles_io)
        def _():
            off = pl.multiple_of(s * n_chunk, 8)
            oout = pl.multiple_of(c * n + s * n_chunk, 8)
            pltpu.sync_copy(acc_sh.at[pl.ds(off, n_chunk)], stage_v)
            pltpu.sync_copy(stage_v, out_hbm.at[pl.ds(oout, n_chunk)])

    return deg_k(dst)


NBUF = 2


def _agg_call(g, src, dst):
    n, w = g.shape
    e = src.shape[0]
    e_per = e // (NC * NS)
    iters = e_per // K
    st_rows = 40
    n_chunk = 1000
    n_tiles_io = n // n_chunk
    st_per = n_chunk // st_rows

    n_groups = iters // NBUF
    n_tail = iters - n_groups * NBUF

    @functools.partial(
        pl.kernel,
        out_type=jax.ShapeDtypeStruct((NC, n, w), jnp.float32),
        mesh=_mesh(),
        scratch_types=(
            [pltpu.VMEM((iters, K), jnp.int32)]
            + [pltpu.VMEM((K,), jnp.int32) for _ in range(NBUF)]
            + [pltpu.VMEM((K, w), jnp.float32) for _ in range(NBUF)]
            + [pltpu.VMEM((st_rows, w), jnp.float32),
               pltpu.VMEM_SHARED((n, w), jnp.float32)]
            + [pltpu.SemaphoreType.DMA for _ in range(NBUF)]
        ),
    )
    def agg_k(g_hbm, src_hbm, dst_hbm, out_hbm,
              dstall_v, i0, i1, r0, r1,
              stage_v, acc_sh, s0, s1):
        idxs = [i0, i1]
        rows = [r0, r1]
        gsem = [s0, s1]
        c = lax.axis_index("c")
        s = lax.axis_index("s")
        wid = c * NS + s
        base = wid * e_per

        pltpu.sync_copy(dst_hbm.at[wid], dstall_v)

        @pl.when(s < n_tiles_io)
        def _():
            zero = jnp.zeros((16,), jnp.float32)

            def zrow(i, carry):
                for j in range(w // 16):
                    stage_v[i, pl.ds(j * 16, 16)] = zero
                return carry

            lax.fori_loop(0, st_rows, zrow, 0)
            for j in range(st_per):
                off = pl.multiple_of(s * n_chunk + j * st_rows, 8)
                pltpu.sync_copy(stage_v, acc_sh.at[pl.ds(off, st_rows)])

        plsc.subcore_barrier()

        for b in range(NBUF):
            off = pl.multiple_of(base + b * K, 8)
            pltpu.sync_copy(src_hbm.at[pl.ds(off, K)], idxs[b])
            pltpu.async_copy(g_hbm.at[idxs[b]], rows[b], gsem[b])

        def step(i, b):
            pltpu.make_async_copy(g_hbm.at[idxs[b]], rows[b], gsem[b]).wait()
            pltpu.sync_copy(rows[b], acc_sh.at[dstall_v.at[i]], add=True)

            @pl.when(i + NBUF < iters)
            def _():
                off = pl.multiple_of(base + (i + NBUF) * K, 8)
                pltpu.sync_copy(src_hbm.at[pl.ds(off, K)], idxs[b])
                pltpu.async_copy(g_hbm.at[idxs[b]], rows[b], gsem[b])

        def group(gi, carry):
            for b in range(NBUF):
                step(gi * NBUF + b, b)
            return carry

        lax.fori_loop(0, n_groups, group, 0)
        for t in range(n_tail):
            step(n_groups * NBUF + t, t)
        plsc.subcore_barrier()

        @pl.when(s < n_tiles_io)
        def _():
            for j in range(st_per):
                off = pl.multiple_of(s * n_chunk + j * st_rows, 8)
                pltpu.sync_copy(acc_sh.at[pl.ds(off, st_rows)], stage_v)
                pltpu.sync_copy(stage_v, out_hbm.at[c, pl.ds(off, st_rows)])

    dst3 = dst.reshape(NC * NS, iters, K)
    return agg_k(g, src, dst3)


_BR = 1000


def _tc_first(x, w0, dega, degb):
    n, d = x.shape
    h = w0.shape[1]

    def body(x_ref, w_ref, da_ref, db_ref, h_ref, g_ref, dv_ref):
        deg = da_ref[...] + db_ref[...] + 2.0
        dinv = lax.rsqrt(deg)
        hh = jnp.dot(x_ref[...], w_ref[...], preferred_element_type=jnp.float32)
        h_ref[...] = hh
        g_ref[...] = hh * dinv
        dv_ref[...] = dinv

    return pl.pallas_call(
        body,
        grid=(n // _BR,),
        in_specs=[
            pl.BlockSpec((_BR, d), lambda i: (i, 0)),
            pl.BlockSpec((d, h), lambda i: (0, 0)),
            pl.BlockSpec((_BR, 1), lambda i: (i, 0)),
            pl.BlockSpec((_BR, 1), lambda i: (i, 0)),
        ],
        out_specs=[
            pl.BlockSpec((_BR, h), lambda i: (i, 0)),
            pl.BlockSpec((_BR, h), lambda i: (i, 0)),
            pl.BlockSpec((_BR, 1), lambda i: (i, 0)),
        ],
        out_shape=[
            jax.ShapeDtypeStruct((n, h), jnp.float32),
            jax.ShapeDtypeStruct((n, h), jnp.float32),
            jax.ShapeDtypeStruct((n, 1), jnp.float32),
        ],
    )(x, w0, dega, degb)


def _tc_mid(acca, accb, hprev, dinv, b, w):
    n, hp = hprev.shape
    hn = w.shape[1]

    def body(aa_ref, ab_ref, hp_ref, dv_ref, b_ref, w_ref, h_ref, g_ref):
        dinv = dv_ref[...]
        acc = aa_ref[...] + ab_ref[...]
        xn = acc * dinv + (2.0 * dinv * dinv) * hp_ref[...] + b_ref[...]
        xn = jnp.maximum(xn, 0.0)
        hh = jnp.dot(xn, w_ref[...], preferred_element_type=jnp.float32)
        h_ref[...] = hh
        g_ref[...] = hh * dinv

    return pl.pallas_call(
        body,
        grid=(n // _BR,),
        in_specs=[
            pl.BlockSpec((_BR, hp), lambda i: (i, 0)),
            pl.BlockSpec((_BR, hp), lambda i: (i, 0)),
            pl.BlockSpec((_BR, hp), lambda i: (i, 0)),
            pl.BlockSpec((_BR, 1), lambda i: (i, 0)),
            pl.BlockSpec((1, hp), lambda i: (0, 0)),
            pl.BlockSpec((hp, hn), lambda i: (0, 0)),
        ],
        out_specs=[
            pl.BlockSpec((_BR, hn), lambda i: (i, 0)),
            pl.BlockSpec((_BR, hn), lambda i: (i, 0)),
        ],
        out_shape=[
            jax.ShapeDtypeStruct((n, hn), jnp.float32),
            jax.ShapeDtypeStruct((n, hn), jnp.float32),
        ],
    )(acca, accb, hprev, dinv, b, w)


def _tc_final(acca, accb, hprev, dinv, b):
    n, hp = hprev.shape

    def body(aa_ref, ab_ref, hp_ref, dv_ref, b_ref, o_ref):
        dinv = dv_ref[...]
        acc = aa_ref[...] + ab_ref[...]
        xn = acc * dinv + (2.0 * dinv * dinv) * hp_ref[...] + b_ref[...]
        o_ref[...] = jnp.tanh(xn)

    return pl.pallas_call(
        body,
        grid=(n // _BR,),
        in_specs=[
            pl.BlockSpec((_BR, hp), lambda i: (i, 0)),
            pl.BlockSpec((_BR, hp), lambda i: (i, 0)),
            pl.BlockSpec((_BR, hp), lambda i: (i, 0)),
            pl.BlockSpec((_BR, 1), lambda i: (i, 0)),
            pl.BlockSpec((1, hp), lambda i: (0, 0)),
        ],
        out_specs=pl.BlockSpec((_BR, hp), lambda i: (i, 0)),
        out_shape=jax.ShapeDtypeStruct((n, hp), jnp.float32),
    )(acca, accb, hprev, dinv, b)


def kernel(x, edge_index, W0, b0, W1, b1, W2, b2):
    n, d = x.shape
    h = W0.shape[1]
    c_out = W2.shape[1]
    c_pad = 128

    src = edge_index[0].astype(jnp.int32)
    dst = edge_index[1].astype(jnp.int32)

    b0r = b0.reshape(1, h)
    b1r = b1.reshape(1, h)
    w2p = jnp.zeros((h, c_pad), jnp.float32).at[:, :c_out].set(W2)
    b2p = jnp.zeros((1, c_pad), jnp.float32).at[0, :c_out].set(b2)

    degp = _deg_call(dst, n)
    dega = degp[:n].reshape(n, 1)
    degb = degp[n:].reshape(n, 1)

    h0, g0, dinv = _tc_first(x, W0, dega, degb)
    acc0 = _agg_call(g0, src, dst)
    h1, g1 = _tc_mid(acc0[0], acc0[1], h0, dinv, b0r, W1)
    acc1 = _agg_call(g1, src, dst)
    h2, g2 = _tc_mid(acc1[0], acc1[1], h1, dinv, b1r, w2p)
    acc2 = _agg_call(g2, src, dst)
    out = _tc_final(acc2[0], acc2[1], h2, dinv, b2p)
    return out[:, :c_out]


# speedup vs baseline: 19.2889x; 1.0256x over previous
import functools

import jax
import jax.numpy as jnp
from jax import lax
from jax.experimental import pallas as pl
from jax.experimental.pallas import tpu as pltpu
from jax.experimental.pallas import tpu_sc as plsc

NC = 2
NS = 16
K = 80


def _mesh():
    return plsc.VectorSubcoreMesh(core_axis_name="c", subcore_axis_name="s")


def _deg_call(dst, n):
    e = dst.shape[0]
    e_per = e // (NC * NS)
    iters = e_per // K
    n_chunk = 1000
    n_tiles_io = n // n_chunk

    @functools.partial(
        pl.kernel,
        out_type=jax.ShapeDtypeStruct((NC * n,), jnp.float32),
        mesh=_mesh(),
        scratch_types=[
            pltpu.VMEM((e_per // K, K), jnp.int32),
            pltpu.VMEM((K,), jnp.float32),
            pltpu.VMEM((n_chunk,), jnp.float32),
            pltpu.VMEM_SHARED((n,), jnp.float32),
        ],
    )
    def deg_k(dst_hbm, out_hbm, dstall_v, ones_v, stage_v, acc_sh):
        c = lax.axis_index("c")
        s = lax.axis_index("s")
        wid = c * NS + s

        pltpu.sync_copy(dst_hbm.at[wid], dstall_v)
        for i in range(K // 16):
            ones_v[pl.ds(i * 16, 16)] = jnp.full((16,), 1.0, jnp.float32)

        @pl.when(s < n_tiles_io)
        def _():
            zero = jnp.zeros((16,), jnp.float32)
            for i in range(n_chunk // 16):
                stage_v[pl.ds(i * 16, 16)] = zero
            if n_chunk % 16:
                stage_v[pl.ds(n_chunk - 16, 16)] = zero
            off = pl.multiple_of(s * n_chunk, 8)
            pltpu.sync_copy(stage_v, acc_sh.at[pl.ds(off, n_chunk)])

        plsc.subcore_barrier()

        def body(i, carry):
            pltpu.sync_copy(ones_v, acc_sh.at[dstall_v.at[i]], add=True)
            return carry

        lax.fori_loop(0, iters, body, 0)
        plsc.subcore_barrier()

        @pl.when(s < n_tiles_io)
        def _():
            off = pl.multiple_of(s * n_chunk, 8)
            oout = pl.multiple_of(c * n + s * n_chunk, 8)
            pltpu.sync_copy(acc_sh.at[pl.ds(off, n_chunk)], stage_v)
            pltpu.sync_copy(stage_v, out_hbm.at[pl.ds(oout, n_chunk)])

    return deg_k(dst.reshape(NC * NS, e_per // K, K))


NBUF = 3


def _agg_call(g, src, dst):
    n, w = g.shape
    e = src.shape[0]
    e_per = e // (NC * NS)
    iters = e_per // K
    st_rows = 8
    n_chunk = 1000
    n_tiles_io = n // n_chunk
    st_per = n_chunk // st_rows

    n_groups = iters // NBUF
    n_tail = iters - n_groups * NBUF

    @functools.partial(
        pl.kernel,
        out_type=jax.ShapeDtypeStruct((NC, n, w), jnp.float32),
        mesh=_mesh(),
        scratch_types=(
            [pltpu.VMEM((iters, K), jnp.int32)]
            + [pltpu.VMEM((K,), jnp.int32) for _ in range(NBUF)]
            + [pltpu.VMEM((K, w), jnp.float32) for _ in range(NBUF)]
            + [pltpu.VMEM((st_rows, w), jnp.float32),
               pltpu.VMEM_SHARED((n, w), jnp.float32)]
            + [pltpu.SemaphoreType.DMA for _ in range(NBUF)]
        ),
    )
    def agg_k(g_hbm, src_hbm, dst_hbm, out_hbm,
              dstall_v, i0, i1, i2, r0, r1, r2,
              stage_v, acc_sh, s0, s1, s2):
        idxs = [i0, i1, i2]
        rows = [r0, r1, r2]
        gsem = [s0, s1, s2]
        c = lax.axis_index("c")
        s = lax.axis_index("s")
        wid = c * NS + s
        base = wid * e_per

        pltpu.sync_copy(dst_hbm.at[wid], dstall_v)

        @pl.when(s < n_tiles_io)
        def _():
            zero = jnp.zeros((16,), jnp.float32)

            def zrow(i, carry):
                for j in range(w // 16):
                    stage_v[i, pl.ds(j * 16, 16)] = zero
                return carry

            lax.fori_loop(0, st_rows, zrow, 0)
            for j in range(st_per):
                off = pl.multiple_of(s * n_chunk + j * st_rows, 8)
                pltpu.sync_copy(stage_v, acc_sh.at[pl.ds(off, st_rows)])

        plsc.subcore_barrier()

        for b in range(NBUF):
            off = pl.multiple_of(base + b * K, 8)
            pltpu.sync_copy(src_hbm.at[pl.ds(off, K)], idxs[b])
            pltpu.async_copy(g_hbm.at[idxs[b]], rows[b], gsem[b])

        def step(i, b):
            pltpu.make_async_copy(g_hbm.at[idxs[b]], rows[b], gsem[b]).wait()
            pltpu.sync_copy(rows[b], acc_sh.at[dstall_v.at[i]], add=True)

            @pl.when(i + NBUF < iters)
            def _():
                off = pl.multiple_of(base + (i + NBUF) * K, 8)
                pltpu.sync_copy(src_hbm.at[pl.ds(off, K)], idxs[b])
                pltpu.async_copy(g_hbm.at[idxs[b]], rows[b], gsem[b])

        def group(gi, carry):
            for b in range(NBUF):
                step(gi * NBUF + b, b)
            return carry

        lax.fori_loop(0, n_groups, group, 0)
        for t in range(n_tail):
            step(n_groups * NBUF + t, t)
        plsc.subcore_barrier()

        @pl.when(s < n_tiles_io)
        def _():
            for j in range(st_per):
                off = pl.multiple_of(s * n_chunk + j * st_rows, 8)
                pltpu.sync_copy(acc_sh.at[pl.ds(off, st_rows)], stage_v)
                pltpu.sync_copy(stage_v, out_hbm.at[c, pl.ds(off, st_rows)])

    dst3 = dst.reshape(NC * NS, iters, K)
    return agg_k(g, src, dst3)


_BR = 1000


def _tc_first(x, w0, dega, degb):
    n, d = x.shape
    h = w0.shape[1]

    def body(x_ref, w_ref, da_ref, db_ref, h_ref, g_ref, dv_ref):
        deg = da_ref[...] + db_ref[...] + 2.0
        dinv = lax.rsqrt(deg)
        hh = jnp.dot(x_ref[...], w_ref[...], preferred_element_type=jnp.float32)
        h_ref[...] = hh
        g_ref[...] = hh * dinv
        dv_ref[...] = dinv

    return pl.pallas_call(
        body,
        grid=(n // _BR,),
        in_specs=[
            pl.BlockSpec((_BR, d), lambda i: (i, 0)),
            pl.BlockSpec((d, h), lambda i: (0, 0)),
            pl.BlockSpec((_BR, 1), lambda i: (i, 0)),
            pl.BlockSpec((_BR, 1), lambda i: (i, 0)),
        ],
        out_specs=[
            pl.BlockSpec((_BR, h), lambda i: (i, 0)),
            pl.BlockSpec((_BR, h), lambda i: (i, 0)),
            pl.BlockSpec((_BR, 1), lambda i: (i, 0)),
        ],
        out_shape=[
            jax.ShapeDtypeStruct((n, h), jnp.float32),
            jax.ShapeDtypeStruct((n, h), jnp.float32),
            jax.ShapeDtypeStruct((n, 1), jnp.float32),
        ],
    )(x, w0, dega, degb)


def _tc_mid(acca, accb, hprev, dinv, b, w):
    n, hp = hprev.shape
    hn = w.shape[1]

    def body(aa_ref, ab_ref, hp_ref, dv_ref, b_ref, w_ref, h_ref, g_ref):
        dinv = dv_ref[...]
        acc = aa_ref[...] + ab_ref[...]
        xn = acc * dinv + (2.0 * dinv * dinv) * hp_ref[...] + b_ref[...]
        xn = jnp.maximum(xn, 0.0)
        hh = jnp.dot(xn, w_ref[...], preferred_element_type=jnp.float32)
        h_ref[...] = hh
        g_ref[...] = hh * dinv

    return pl.pallas_call(
        body,
        grid=(n // _BR,),
        in_specs=[
            pl.BlockSpec((_BR, hp), lambda i: (i, 0)),
            pl.BlockSpec((_BR, hp), lambda i: (i, 0)),
            pl.BlockSpec((_BR, hp), lambda i: (i, 0)),
            pl.BlockSpec((_BR, 1), lambda i: (i, 0)),
            pl.BlockSpec((1, hp), lambda i: (0, 0)),
            pl.BlockSpec((hp, hn), lambda i: (0, 0)),
        ],
        out_specs=[
            pl.BlockSpec((_BR, hn), lambda i: (i, 0)),
            pl.BlockSpec((_BR, hn), lambda i: (i, 0)),
        ],
        out_shape=[
            jax.ShapeDtypeStruct((n, hn), jnp.float32),
            jax.ShapeDtypeStruct((n, hn), jnp.float32),
        ],
    )(acca, accb, hprev, dinv, b, w)


def _tc_final(acca, accb, hprev, dinv, b):
    n, hp = hprev.shape

    def body(aa_ref, ab_ref, hp_ref, dv_ref, b_ref, o_ref):
        dinv = dv_ref[...]
        acc = aa_ref[...] + ab_ref[...]
        xn = acc * dinv + (2.0 * dinv * dinv) * hp_ref[...] + b_ref[...]
        o_ref[...] = jnp.tanh(xn)

    return pl.pallas_call(
        body,
        grid=(n // _BR,),
        in_specs=[
            pl.BlockSpec((_BR, hp), lambda i: (i, 0)),
            pl.BlockSpec((_BR, hp), lambda i: (i, 0)),
            pl.BlockSpec((_BR, hp), lambda i: (i, 0)),
            pl.BlockSpec((_BR, 1), lambda i: (i, 0)),
            pl.BlockSpec((1, hp), lambda i: (0, 0)),
        ],
        out_specs=pl.BlockSpec((_BR, hp), lambda i: (i, 0)),
        out_shape=jax.ShapeDtypeStruct((n, hp), jnp.float32),
    )(acca, accb, hprev, dinv, b)


def kernel(x, edge_index, W0, b0, W1, b1, W2, b2):
    n, d = x.shape
    h = W0.shape[1]
    c_out = W2.shape[1]
    c_pad = 128

    src = edge_index[0].astype(jnp.int32)
    dst = edge_index[1].astype(jnp.int32)

    b0r = b0.reshape(1, h)
    b1r = b1.reshape(1, h)
    w2p = jnp.zeros((h, c_pad), jnp.float32).at[:, :c_out].set(W2)
    b2p = jnp.zeros((1, c_pad), jnp.float32).at[0, :c_out].set(b2)

    degp = _deg_call(dst, n)
    dega = degp[:n].reshape(n, 1)
    degb = degp[n:].reshape(n, 1)

    h0, g0, dinv = _tc_first(x, W0, dega, degb)
    acc0 = _agg_call(g0, src, dst)
    h1, g1 = _tc_mid(acc0[0], acc0[1], h0, dinv, b0r, W1)
    acc1 = _agg_call(g1, src, dst)
    h2, g2 = _tc_mid(acc1[0], acc1[1], h1, dinv, b1r, w2p)
    acc2 = _agg_call(g2, src, dst)
    out = _tc_final(acc2[0], acc2[1], h2, dinv, b2p)
    return out[:, :c_out]
